# baseline (speedup 1.0000x reference)
import jax
import jax.numpy as jnp
from jax import lax
from jax.experimental import pallas as pl
from jax.experimental.pallas import tpu as pltpu


def _kernel(sidx_ref, bidx_ref, oidx_ref, nf_ref, seq_ref, tmpl_ref,
            edge_ref, refer_ref, out_ref):
    del sidx_ref, bidx_ref, oidx_ref
    i = pl.program_id(0)
    in_copy_phase = i < nf_ref[0]

    @pl.when(in_copy_phase)
    def _():
        out_ref[...] = seq_ref[...]

    @pl.when(jnp.logical_not(in_copy_phase))
    def _():
        e = edge_ref[...]
        t = tmpl_ref[...]
        r = refer_ref[...]
        out_ref[...] = jax.nn.sigmoid(t) * e + r * (1.0 - e)


def kernel(seq, refer_seq, attack_index, edge_mask, template):
    _, F, H, W = seq.shape
    K = attack_index.shape[0]
    ai = attack_index.astype(jnp.int32)

    pos = jnp.full((F,), -1, jnp.int32).at[ai].set(
        jnp.arange(K, dtype=jnp.int32), mode="drop")
    arange_f = jnp.arange(F, dtype=jnp.int32)
    not_attacked = pos < 0
    last_na = lax.cummax(jnp.where(not_attacked, arange_f, -1))
    first_na = jnp.argmax(not_attacked).astype(jnp.int32)
    cfix = jnp.where(last_na < 0, first_na, last_na)

    sidx = jnp.concatenate([cfix, jnp.full((K,), cfix[F - 1], jnp.int32)])
    bidx = jnp.concatenate([jnp.zeros((F,), jnp.int32),
                            jnp.arange(K, dtype=jnp.int32)])
    oidx = jnp.concatenate([cfix, ai])
    nf = jnp.full((1,), F, jnp.int32)

    grid_spec = pltpu.PrefetchScalarGridSpec(
        num_scalar_prefetch=4,
        grid=(F + K,),
        in_specs=[
            pl.BlockSpec((1, 1, H, W),
                         lambda i, sidx, bidx, oidx, nf: (0, sidx[i], 0, 0)),
            pl.BlockSpec((1, 1, H, W),
                         lambda i, sidx, bidx, oidx, nf: (0, bidx[i], 0, 0)),
            pl.BlockSpec((1, 1, H, W),
                         lambda i, sidx, bidx, oidx, nf: (0, bidx[i], 0, 0)),
            pl.BlockSpec((1, 1, H, W),
                         lambda i, sidx, bidx, oidx, nf: (0, bidx[i], 0, 0)),
        ],
        out_specs=pl.BlockSpec((1, 1, H, W),
                               lambda i, sidx, bidx, oidx, nf: (0, oidx[i], 0, 0)),
    )
    return pl.pallas_call(
        _kernel,
        grid_spec=grid_spec,
        out_shape=jax.ShapeDtypeStruct(seq.shape, seq.dtype),
    )(sidx, bidx, oidx, nf, seq, template, edge_mask, refer_seq)
